# baseline (speedup 1.0000x reference)
import jax
import jax.numpy as jnp
from jax import lax
from jax.experimental import pallas as pl
from jax.experimental.pallas import tpu as pltpu
from jax.experimental.pallas import tpu_sc as plsc

B, S, H = 4, 2048, 768
N = B * S
EPS = 1e-12

NC, NS, L = 2, 16, 16
NW = NC * NS
TOK_PER_W = N // NW
CHUNK = 8
NCHUNK = TOK_PER_W // CHUNK
NPAIR = NCHUNK // 2
HV = H // L
TYPES, MAXPOS, MAXSEG, MAXSPK = 2, 2048, 64, 16
SPK_OFF = MAXSEG
TYPE_OFF = MAXSEG + MAXSPK
NROWS = MAXSEG + MAXSPK + TYPES


def _rsqrt(x):
    xh = 0.5 * x
    i = lax.bitcast_convert_type(x, jnp.int32)
    i = jnp.int32(0x5F3759DF) - (i >> 1)
    y = lax.bitcast_convert_type(i, jnp.float32)
    y = y * (1.5 - xh * y * y)
    y = y * (1.5 - xh * y * y)
    y = y * (1.5 - xh * y * y)
    return y


def _body(ids_w, ids_p, ids_c,
          word_hbm, pos_hbm, combo_hbm,
          gamma_hbm, beta_hbm, out_hbm,
          iw, ip, ic, gbuf, bbuf,
          bw0, bw1, bp0, bp1, bc0, bc1, ob0, ob1,
          semg0, semg1, semo0, semo1, sems):
    sid = lax.axis_index("s")
    wid = sid * NC + lax.axis_index("c")
    rbase = wid * NCHUNK

    staged = ((gamma_hbm, gbuf), (beta_hbm, bbuf),
              (ids_w.at[pl.ds(rbase, NCHUNK)], iw),
              (ids_p.at[pl.ds(rbase, NCHUNK)], ip),
              (ids_c.at[pl.ds(rbase, NCHUNK)], ic))
    for src, dst in staged:
        pltpu.async_copy(src, dst, sems)
    for src, dst in staged:
        pltpu.make_async_copy(src, dst, sems).wait()

    def fire(c, bw, bp, bc, semg):
        pltpu.async_copy(word_hbm.at[iw.at[c]], bw, semg)
        pltpu.async_copy(pos_hbm.at[ip.at[c]], bp, semg)
        pltpu.async_copy(combo_hbm.at[ic.at[c]], bc, semg)

    def drain(c, bw, bp, bc, semg):
        pltpu.make_async_copy(word_hbm.at[iw.at[c]], bw, semg).wait()
        pltpu.make_async_copy(pos_hbm.at[ip.at[c]], bp, semg).wait()
        pltpu.make_async_copy(combo_hbm.at[ic.at[c]], bc, semg).wait()

    fire(0, bw0, bp0, bc0, semg0)
    fire(1, bw1, bp1, bc1, semg1)

    def compute_chunk(bw, bp, bc, ob):
        def row_body(r, carry):
            s = jnp.zeros((L,), jnp.float32)
            ss = jnp.zeros((L,), jnp.float32)
            for j in range(HV):
                hs = pl.ds(j * L, L)
                v = (bw[r, hs] + bp[r, hs] + bc[r, hs]
                     + bc[r + CHUNK, hs] + bc[r + 2 * CHUNK, hs])
                ob[r, hs] = v
                s = s + v
                ss = ss + v * v
            mean = lax.reduce_sum_p.bind(s, axes=(0,)) * (1.0 / H)
            msq = lax.reduce_sum_p.bind(ss, axes=(0,)) * (1.0 / H)
            rstd = _rsqrt(msq - mean * mean + EPS)
            for j in range(HV):
                hs = pl.ds(j * L, L)
                ob[r, hs] = (ob[r, hs] - mean) * rstd * gbuf[hs] + bbuf[hs]
            return carry
        lax.fori_loop(0, CHUNK, row_body, 0)

    def pair_body(i, carry):
        for c, bw, bp, bc, ob, semg, semo in (
                (2 * i, bw0, bp0, bc0, ob0, semg0, semo0),
                (2 * i + 1, bw1, bp1, bc1, ob1, semg1, semo1)):
            osl = pl.ds((rbase + c) * CHUNK, CHUNK)
            drain(c, bw, bp, bc, semg)

            @pl.when(i > 0)
            def _():
                pltpu.make_async_copy(ob, out_hbm.at[osl], semo).wait()

            compute_chunk(bw, bp, bc, ob)
            pltpu.async_copy(ob, out_hbm.at[osl], semo)

            @pl.when(i < NPAIR - 1)
            def _():
                fire(c + 2, bw, bp, bc, semg)
        return carry

    lax.fori_loop(0, NPAIR, pair_body, 0)

    pltpu.make_async_copy(
        ob0, out_hbm.at[pl.ds((rbase + NCHUNK - 2) * CHUNK, CHUNK)],
        semo0).wait()
    pltpu.make_async_copy(
        ob1, out_hbm.at[pl.ds((rbase + NCHUNK - 1) * CHUNK, CHUNK)],
        semo1).wait()


@jax.jit
def _run(ids_w, ids_p, ids_c,
         word_emb, pos_emb, combo_emb, ln_gamma, ln_beta):
    mesh = plsc.VectorSubcoreMesh(core_axis_name="c", subcore_axis_name="s",
                                  num_cores=NC, num_subcores=NS)
    f = pl.kernel(
        _body,
        out_type=jax.ShapeDtypeStruct((N, H), jnp.float32),
        mesh=mesh,
        scratch_types=[
            pltpu.VMEM((NCHUNK, CHUNK), jnp.int32),
            pltpu.VMEM((NCHUNK, CHUNK), jnp.int32),
            pltpu.VMEM((NCHUNK, 3 * CHUNK), jnp.int32),
            pltpu.VMEM((H,), jnp.float32),
            pltpu.VMEM((H,), jnp.float32),
            pltpu.VMEM((CHUNK, H), jnp.float32),
            pltpu.VMEM((CHUNK, H), jnp.float32),
            pltpu.VMEM((CHUNK, H), jnp.float32),
            pltpu.VMEM((CHUNK, H), jnp.float32),
            pltpu.VMEM((3 * CHUNK, H), jnp.float32),
            pltpu.VMEM((3 * CHUNK, H), jnp.float32),
            pltpu.VMEM((CHUNK, H), jnp.float32),
            pltpu.VMEM((CHUNK, H), jnp.float32),
            pltpu.SemaphoreType.DMA,
            pltpu.SemaphoreType.DMA,
            pltpu.SemaphoreType.DMA,
            pltpu.SemaphoreType.DMA,
            pltpu.SemaphoreType.DMA,
        ],
        compiler_params=pltpu.CompilerParams(needs_layout_passes=False),
        name="wsw_embed_ln",
    )
    return f(ids_w, ids_p, ids_c, word_emb, pos_emb, combo_emb,
             ln_gamma, ln_beta)


def kernel(input_ids, token_type_ids, position_ids, segment_ids, speaker_ids,
           word_emb, type_emb, pos_emb, seg_emb, spk_emb, ln_gamma, ln_beta):
    ids_w = input_ids.reshape(N // CHUNK, CHUNK).astype(jnp.int32)
    ids_p = position_ids.reshape(N // CHUNK, CHUNK).astype(jnp.int32)
    combo = jnp.stack(
        [segment_ids.reshape(N // CHUNK, CHUNK).astype(jnp.int32),
         speaker_ids.reshape(N // CHUNK, CHUNK).astype(jnp.int32) + SPK_OFF,
         token_type_ids.reshape(N // CHUNK, CHUNK).astype(jnp.int32)
         + TYPE_OFF],
        axis=1).reshape(N // CHUNK, 3 * CHUNK)
    combo_emb = jnp.concatenate([seg_emb, spk_emb, type_emb], axis=0)
    out = _run(ids_w, ids_p, combo, word_emb, pos_emb, combo_emb,
               ln_gamma, ln_beta)
    return out.reshape(B, S, H)
